# baseline (speedup 1.0000x reference)
import functools

import jax
import jax.numpy as jnp
import numpy as np
from jax import lax
from jax.experimental import pallas as pl
from jax.experimental.pallas import tpu as pltpu
from jax.experimental.pallas import tpu_sc as plsc

N = 10000
E = 160000
D = 128
NUM_RBF = 96
NUM_SPECIES = 5
D_EDGE = 16
CUTOFF = 5.0

NPAD = 10240
ROWS_PER_TILE = NPAD // 16

NC, NS, NW = 2, 16, 32
EDGES_PER_TILE = E // NW
CHUNK = 128
N_FULL = EDGES_PER_TILE // CHUNK
TAIL = EDGES_PER_TILE - N_FULL * CHUNK
BOUNCE = CHUNK

BE = 2000
BN = 2000


def _sc_gather_body(src_hbm, dst_hbm, species_hbm, sps_hbm, spd_hbm,
                    spec_v, idx_v, out_v, idx_t, out_t):
    wid = lax.axis_index("s") * NC + lax.axis_index("c")
    base = wid * EDGES_PER_TILE

    pltpu.sync_copy(species_hbm, spec_v)

    def one(edge_hbm, out_hbm, off):
        pltpu.sync_copy(edge_hbm.at[pl.ds(off, CHUNK)], idx_v)
        for j in range(CHUNK // 16):
            idx16 = idx_v[pl.ds(j * 16, 16)]
            out_v[pl.ds(j * 16, 16)] = plsc.load_gather(spec_v, [idx16])
        pltpu.sync_copy(out_v, out_hbm.at[pl.ds(off, CHUNK)])

    for i in range(N_FULL):
        one(src_hbm, sps_hbm, base + i * CHUNK)
        one(dst_hbm, spd_hbm, base + i * CHUNK)

    off = base + N_FULL * CHUNK

    def tail(edge_hbm, out_hbm):
        idx_t[...] = jnp.zeros((16,), jnp.int32)
        pltpu.sync_copy(edge_hbm.at[pl.ds(off, TAIL)], idx_t.at[pl.ds(0, TAIL)])
        out_t[...] = plsc.load_gather(spec_v, [idx_t[...]])
        pltpu.sync_copy(out_t.at[pl.ds(0, TAIL)], out_hbm.at[pl.ds(off, TAIL)])

    tail(src_hbm, sps_hbm)
    tail(dst_hbm, spd_hbm)


def _sc_gather(src, dst, species):
    k = pl.kernel(
        _sc_gather_body,
        mesh=plsc.VectorSubcoreMesh(core_axis_name="c", subcore_axis_name="s"),
        out_type=(jax.ShapeDtypeStruct((E,), jnp.int32),
                  jax.ShapeDtypeStruct((E,), jnp.int32)),
        scratch_types=[
            pltpu.VMEM((N,), jnp.int32),
            pltpu.VMEM((CHUNK,), jnp.int32),
            pltpu.VMEM((CHUNK,), jnp.int32),
            pltpu.VMEM((16,), jnp.int32),
            pltpu.VMEM((16,), jnp.int32),
        ],
    )
    return k(src, dst, species)


def _edge_body(sps_ref, spd_ref, eattr_ref, evec_ref,
               wer_ref, wea_ref, be_ref, q8_ref, k8_ref, vm8_ref, u8_ref,
               t8_ref, msg_ref):
    sps_row = sps_ref[0, :, :]
    spd_row = spd_ref[0, :, :]
    evec = evec_ref[:, :]
    elen2 = jnp.sum(evec * evec, axis=1, keepdims=True)
    elen = jnp.sqrt(elen2)

    lane96 = lax.broadcasted_iota(jnp.int32, (BE, NUM_RBF), 1)
    centers = lane96.astype(jnp.float32) * (CUTOFF / (NUM_RBF - 1))
    rbf = jnp.exp(-10.0 * (elen - centers) ** 2)

    pre = (jnp.dot(rbf, wer_ref[:, :], preferred_element_type=jnp.float32, precision=lax.Precision.HIGHEST)
           + jnp.dot(eattr_ref[:, :], wea_ref[:, :],
                     preferred_element_type=jnp.float32, precision=lax.Precision.HIGHEST)
           + be_ref[:, :])
    ea = pre * jax.nn.sigmoid(pre)

    sub8 = lax.broadcasted_iota(jnp.int32, (8, BE), 0)
    p_st = (sps_row == sub8).astype(jnp.float32)
    p_dt = (spd_row == sub8).astype(jnp.float32)

    dimnum = (((0,), (0,)), ((), ()))
    qd = lax.dot_general(p_dt, q8_ref[:, :], dimnum,
                         preferred_element_type=jnp.float32, precision=lax.Precision.HIGHEST)
    ks = lax.dot_general(p_st, k8_ref[:, :], dimnum,
                         preferred_element_type=jnp.float32, precision=lax.Precision.HIGHEST)
    vm = lax.dot_general(p_st, vm8_ref[:, :], dimnum,
                         preferred_element_type=jnp.float32, precision=lax.Precision.HIGHEST)

    logit = jnp.sum(ea * qd * ks, axis=1, keepdims=True) / np.sqrt(D)
    cut = jnp.where(elen < CUTOFF,
                    0.5 * (jnp.cos(jnp.pi * elen / CUTOFF) + 1.0), 0.0)
    attn = logit * jax.nn.sigmoid(logit) * cut

    msg_ref[:, :D] = vm * ea * attn

    ehat = evec / (elen + 1e-8)
    hat16 = jnp.dot(ehat, u8_ref[:3, :],
                    preferred_element_type=jnp.float32, precision=lax.Precision.HIGHEST)
    p16 = lax.dot_general(p_st, t8_ref[:, :], dimnum,
                          preferred_element_type=jnp.float32, precision=lax.Precision.HIGHEST)
    msg_ref[:, D:] = hat16 * p16 * attn


def _tc_edge(sps, spd, edge_attr, edge_vec, wer, wea, be_row, q8, k8, vm8,
             u8, t8):
    grid = E // BE
    return pl.pallas_call(
        _edge_body,
        grid=(grid,),
        in_specs=[
            pl.BlockSpec((1, 1, BE), lambda i: (i, 0, 0)),
            pl.BlockSpec((1, 1, BE), lambda i: (i, 0, 0)),
            pl.BlockSpec((BE, D_EDGE), lambda i: (i, 0)),
            pl.BlockSpec((BE, 3), lambda i: (i, 0)),
            pl.BlockSpec((NUM_RBF, D), lambda i: (0, 0)),
            pl.BlockSpec((D_EDGE, D), lambda i: (0, 0)),
            pl.BlockSpec((1, D), lambda i: (0, 0)),
            pl.BlockSpec((8, D), lambda i: (0, 0)),
            pl.BlockSpec((8, D), lambda i: (0, 0)),
            pl.BlockSpec((8, D), lambda i: (0, 0)),
            pl.BlockSpec((8, 16), lambda i: (0, 0)),
            pl.BlockSpec((8, 16), lambda i: (0, 0)),
        ],
        out_specs=pl.BlockSpec((BE, D + 16), lambda i: (i, 0)),
        out_shape=jax.ShapeDtypeStruct((E, D + 16), jnp.float32),
    )(sps, spd, edge_attr, edge_vec, wer, wea, be_row, q8, k8, vm8, u8, t8)


BS = 2000


def _scat_body(dst_ref, msg_ref, acc_ref):
    pid = pl.program_id(0)

    @pl.when(pid == 0)
    def _():
        acc_ref[...] = jnp.zeros((N, D + 16), jnp.float32)

    def body(i, _):
        d = dst_ref[0, 0, i]
        acc_ref[pl.ds(d, 1), :] += msg_ref[pl.ds(i, 1), :]
        return ()

    lax.fori_loop(0, BS, body, ())


def _tc_scatter(dst3, msg):
    grid = E // BS
    return pl.pallas_call(
        _scat_body,
        grid=(grid,),
        in_specs=[
            pl.BlockSpec((1, 1, BS), lambda i: (i, 0, 0),
                         memory_space=pltpu.SMEM),
            pl.BlockSpec((BS, D + 16), lambda i: (i, 0)),
        ],
        out_specs=pl.BlockSpec((N, D + 16), lambda i: (0, 0)),
        out_shape=jax.ShapeDtypeStruct((N, D + 16), jnp.float32),
    )(dst3, msg)


def _node_body(spe_ref, acc_ref, h8_ref, wo_ref, wd1_ref, bd1_ref,
               wd2_ref, bd2_ref, wg_ref, s2t_ref, t8_ref, sel_ref,
               ns_ref, vec_ref):
    dh = acc_ref[:, :D]
    a16 = acc_ref[:, D:]

    lane8 = lax.broadcasted_iota(jnp.int32, (BN, 8), 1)
    p = (spe_ref[:, :] == lane8).astype(jnp.float32)
    h0 = jnp.dot(p, h8_ref[:, :], preferred_element_type=jnp.float32, precision=lax.Precision.HIGHEST)
    h = h0 + jnp.dot(dh, wo_ref[:, :], preferred_element_type=jnp.float32, precision=lax.Precision.HIGHEST)

    t = jnp.dot(h, wd1_ref[:, :], preferred_element_type=jnp.float32, precision=lax.Precision.HIGHEST) \
        + bd1_ref[:, :]
    t = t * jax.nn.sigmoid(t)
    ns_ref[:, :] = (jnp.sum(t * wd2_ref[:, :], axis=1, keepdims=True)
                    + bd2_ref[:, :])

    gate = jnp.dot(h, wg_ref[:, :], preferred_element_type=jnp.float32, precision=lax.Precision.HIGHEST)
    g8 = jnp.dot(gate, s2t_ref[:, :], preferred_element_type=jnp.float32, precision=lax.Precision.HIGHEST)
    g16 = jnp.dot(g8, t8_ref[:, :], preferred_element_type=jnp.float32, precision=lax.Precision.HIGHEST)
    vec8 = jnp.dot(a16 * g16, sel_ref[:, :],
                   preferred_element_type=jnp.float32, precision=lax.Precision.HIGHEST)
    vec_ref[:, :] = vec8[:, :3]


def _tc_node(spe, acc, h8, wo, wd1, bd1, wd2row, bd2s, wg, s2t, t8,
             sel):
    grid = N // BN
    return pl.pallas_call(
        _node_body,
        grid=(grid,),
        in_specs=[
            pl.BlockSpec((BN, 1), lambda i: (i, 0)),
            pl.BlockSpec((BN, D + 16), lambda i: (i, 0)),
            pl.BlockSpec((8, D), lambda i: (0, 0)),
            pl.BlockSpec((D, D), lambda i: (0, 0)),
            pl.BlockSpec((D, 64), lambda i: (0, 0)),
            pl.BlockSpec((1, 64), lambda i: (0, 0)),
            pl.BlockSpec((1, 64), lambda i: (0, 0)),
            pl.BlockSpec((1, 1), lambda i: (0, 0)),
            pl.BlockSpec((D, D), lambda i: (0, 0)),
            pl.BlockSpec((D, 8), lambda i: (0, 0)),
            pl.BlockSpec((8, 16), lambda i: (0, 0)),
            pl.BlockSpec((16, 8), lambda i: (0, 0)),
        ],
        out_specs=[
            pl.BlockSpec((BN, 1), lambda i: (i, 0)),
            pl.BlockSpec((BN, 3), lambda i: (i, 0)),
        ],
        out_shape=[
            jax.ShapeDtypeStruct((N, 1), jnp.float32),
            jax.ShapeDtypeStruct((N, 3), jnp.float32),
        ],
    )(spe, acc, h8, wo, wd1, bd1, wd2row, bd2s, wg, s2t, t8, sel)


def kernel(species, edge_index, edge_attr, edge_vec, W1, W2, b2, We, be,
           Wq, Wk, Wv, Wo, Wd1, bd1, Wd2, bd2, Wg):
    species = species.astype(jnp.int32)
    src = edge_index[0].astype(jnp.int32)
    dst = edge_index[1].astype(jnp.int32)

    pm = jnp.promote_types
    hp = lambda a, b: jnp.dot(a, b, precision=lax.Precision.HIGHEST)
    h5 = hp(jax.nn.silu(W1), W2) + b2
    h8 = jnp.concatenate([h5, jnp.zeros((3, D), jnp.float32)], axis=0)
    q8 = hp(h8, Wq)
    k8 = hp(h8, Wk)
    vm8 = hp(h8, Wv[:, :D])
    s2_5 = hp(h5, Wv[:, 2 * D:])
    s2t = jnp.concatenate([s2_5.T, jnp.zeros((D, 3), jnp.float32)], axis=1)

    wer = We[:NUM_RBF]
    wea = We[NUM_RBF:]
    be_row = be.reshape(1, D)

    u8 = np.zeros((8, 16), np.float32)
    t8 = np.zeros((8, 16), np.float32)
    sel = np.zeros((16, 8), np.float32)
    for c in range(3):
        u8[c, c * 5:c * 5 + 5] = 1.0
        for b in range(5):
            t8[b, c * 5 + b] = 1.0
            sel[c * 5 + b, c] = 1.0
    u8 = jnp.asarray(u8)
    t8 = jnp.asarray(t8)
    sel = jnp.asarray(sel)

    sps, spd = species[src], species[dst]
    msg = _tc_edge(sps.reshape(E // BE, 1, BE), spd.reshape(E // BE, 1, BE),
                   edge_attr, edge_vec, wer, wea, be_row,
                   q8, k8, vm8, u8, t8)

    acc = _tc_scatter(dst.reshape(E // BS, 1, BS), msg)

    ns, vec = _tc_node(species.reshape(N, 1), acc,
                       h8, Wo, Wd1, bd1.reshape(1, 64), Wd2.reshape(1, 64),
                       bd2.reshape(1, 1), Wg, s2t, t8, sel)
    return ns, vec


# speedup vs baseline: 1.0477x; 1.0477x over previous
import functools

import jax
import jax.numpy as jnp
import numpy as np
from jax import lax
from jax.experimental import pallas as pl
from jax.experimental.pallas import tpu as pltpu
from jax.experimental.pallas import tpu_sc as plsc

N = 10000
E = 160000
D = 128
NUM_RBF = 96
NUM_SPECIES = 5
D_EDGE = 16
CUTOFF = 5.0

NPAD = 10240
ROWS_PER_TILE = NPAD // 16

NC, NS, NW = 2, 16, 32
EDGES_PER_TILE = E // NW
CHUNK = 128
N_FULL = EDGES_PER_TILE // CHUNK
TAIL = EDGES_PER_TILE - N_FULL * CHUNK
BOUNCE = CHUNK

BE = 2000
BN = 2000


def _sc_gather_body(src_hbm, dst_hbm, species_hbm, sps_hbm, spd_hbm,
                    spec_v, idx_v, out_v, idx_t, out_t):
    wid = lax.axis_index("s") * NC + lax.axis_index("c")
    base = wid * EDGES_PER_TILE

    pltpu.sync_copy(species_hbm, spec_v)

    def one(edge_hbm, out_hbm, off):
        pltpu.sync_copy(edge_hbm.at[pl.ds(off, CHUNK)], idx_v)
        for j in range(CHUNK // 16):
            idx16 = idx_v[pl.ds(j * 16, 16)]
            out_v[pl.ds(j * 16, 16)] = plsc.load_gather(spec_v, [idx16])
        pltpu.sync_copy(out_v, out_hbm.at[pl.ds(off, CHUNK)])

    for i in range(N_FULL):
        one(src_hbm, sps_hbm, base + i * CHUNK)
        one(dst_hbm, spd_hbm, base + i * CHUNK)

    off = base + N_FULL * CHUNK

    def tail(edge_hbm, out_hbm):
        idx_t[...] = jnp.zeros((16,), jnp.int32)
        pltpu.sync_copy(edge_hbm.at[pl.ds(off, TAIL)], idx_t.at[pl.ds(0, TAIL)])
        out_t[...] = plsc.load_gather(spec_v, [idx_t[...]])
        pltpu.sync_copy(out_t.at[pl.ds(0, TAIL)], out_hbm.at[pl.ds(off, TAIL)])

    tail(src_hbm, sps_hbm)
    tail(dst_hbm, spd_hbm)


def _sc_gather(src, dst, species):
    k = pl.kernel(
        _sc_gather_body,
        mesh=plsc.VectorSubcoreMesh(core_axis_name="c", subcore_axis_name="s"),
        out_type=(jax.ShapeDtypeStruct((E,), jnp.int32),
                  jax.ShapeDtypeStruct((E,), jnp.int32)),
        scratch_types=[
            pltpu.VMEM((N,), jnp.int32),
            pltpu.VMEM((CHUNK,), jnp.int32),
            pltpu.VMEM((CHUNK,), jnp.int32),
            pltpu.VMEM((16,), jnp.int32),
            pltpu.VMEM((16,), jnp.int32),
        ],
    )
    return k(src, dst, species)


def _edge_body(sps_ref, spd_ref, eattr_ref, evec_ref,
               wer_ref, wea_ref, be_ref, q8_ref, k8_ref, vm8_ref, u8_ref,
               t8_ref, msg_ref, amsg_ref):
    sps_row = sps_ref[0, :, :]
    spd_row = spd_ref[0, :, :]
    evec = evec_ref[:, :]
    elen2 = jnp.sum(evec * evec, axis=1, keepdims=True)
    elen = jnp.sqrt(elen2)

    lane96 = lax.broadcasted_iota(jnp.int32, (BE, NUM_RBF), 1)
    centers = lane96.astype(jnp.float32) * (CUTOFF / (NUM_RBF - 1))
    rbf = jnp.exp(-10.0 * (elen - centers) ** 2)

    pre = (jnp.dot(rbf, wer_ref[:, :], preferred_element_type=jnp.float32, precision=lax.Precision.HIGHEST)
           + jnp.dot(eattr_ref[:, :], wea_ref[:, :],
                     preferred_element_type=jnp.float32, precision=lax.Precision.HIGHEST)
           + be_ref[:, :])
    ea = pre * jax.nn.sigmoid(pre)

    sub8 = lax.broadcasted_iota(jnp.int32, (8, BE), 0)
    p_st = (sps_row == sub8).astype(jnp.float32)
    p_dt = (spd_row == sub8).astype(jnp.float32)

    dimnum = (((0,), (0,)), ((), ()))
    qd = lax.dot_general(p_dt, q8_ref[:, :], dimnum,
                         preferred_element_type=jnp.float32, precision=lax.Precision.HIGHEST)
    ks = lax.dot_general(p_st, k8_ref[:, :], dimnum,
                         preferred_element_type=jnp.float32, precision=lax.Precision.HIGHEST)
    vm = lax.dot_general(p_st, vm8_ref[:, :], dimnum,
                         preferred_element_type=jnp.float32, precision=lax.Precision.HIGHEST)

    logit = jnp.sum(ea * qd * ks, axis=1, keepdims=True) / np.sqrt(D)
    cut = jnp.where(elen < CUTOFF,
                    0.5 * (jnp.cos(jnp.pi * elen / CUTOFF) + 1.0), 0.0)
    attn = logit * jax.nn.sigmoid(logit) * cut

    msg_ref[:, :] = vm * ea * attn

    ehat = evec / (elen + 1e-8)
    hat16 = jnp.dot(ehat, u8_ref[:3, :],
                    preferred_element_type=jnp.float32, precision=lax.Precision.HIGHEST)
    p16 = lax.dot_general(p_st, t8_ref[:, :], dimnum,
                          preferred_element_type=jnp.float32, precision=lax.Precision.HIGHEST)
    amsg_ref[:, :] = hat16 * p16 * attn


def _tc_edge(sps, spd, edge_attr, edge_vec, wer, wea, be_row, q8, k8, vm8,
             u8, t8):
    grid = E // BE
    return pl.pallas_call(
        _edge_body,
        grid=(grid,),
        in_specs=[
            pl.BlockSpec((1, 1, BE), lambda i: (i, 0, 0)),
            pl.BlockSpec((1, 1, BE), lambda i: (i, 0, 0)),
            pl.BlockSpec((BE, D_EDGE), lambda i: (i, 0)),
            pl.BlockSpec((BE, 3), lambda i: (i, 0)),
            pl.BlockSpec((NUM_RBF, D), lambda i: (0, 0)),
            pl.BlockSpec((D_EDGE, D), lambda i: (0, 0)),
            pl.BlockSpec((1, D), lambda i: (0, 0)),
            pl.BlockSpec((8, D), lambda i: (0, 0)),
            pl.BlockSpec((8, D), lambda i: (0, 0)),
            pl.BlockSpec((8, D), lambda i: (0, 0)),
            pl.BlockSpec((8, 16), lambda i: (0, 0)),
            pl.BlockSpec((8, 16), lambda i: (0, 0)),
        ],
        out_specs=[
            pl.BlockSpec((BE, D), lambda i: (i, 0)),
            pl.BlockSpec((BE, 16), lambda i: (i, 0)),
        ],
        out_shape=[
            jax.ShapeDtypeStruct((E, D), jnp.float32),
            jax.ShapeDtypeStruct((E, 16), jnp.float32),
        ],
    )(sps, spd, edge_attr, edge_vec, wer, wea, be_row, q8, k8, vm8, u8, t8)


BS = 2000


def _scat_body(dst_ref, msg_ref, amsg_ref, dh_ref, a16_ref):
    pid = pl.program_id(0)

    @pl.when(pid == 0)
    def _():
        dh_ref[...] = jnp.zeros((N, D), jnp.float32)
        a16_ref[...] = jnp.zeros((N, 16), jnp.float32)

    def body(i, _):
        d = dst_ref[0, 0, i]
        dh_ref[pl.ds(d, 1), :] += msg_ref[pl.ds(i, 1), :]
        a16_ref[pl.ds(d, 1), :] += amsg_ref[pl.ds(i, 1), :]
        return ()

    lax.fori_loop(0, BS, body, ())


def _tc_scatter(dst3, msg, amsg):
    grid = E // BS
    return pl.pallas_call(
        _scat_body,
        grid=(grid,),
        in_specs=[
            pl.BlockSpec((1, 1, BS), lambda i: (i, 0, 0),
                         memory_space=pltpu.SMEM),
            pl.BlockSpec((BS, D), lambda i: (i, 0)),
            pl.BlockSpec((BS, 16), lambda i: (i, 0)),
        ],
        out_specs=[
            pl.BlockSpec((N, D), lambda i: (0, 0)),
            pl.BlockSpec((N, 16), lambda i: (0, 0)),
        ],
        out_shape=[
            jax.ShapeDtypeStruct((N, D), jnp.float32),
            jax.ShapeDtypeStruct((N, 16), jnp.float32),
        ],
    )(dst3, msg, amsg)


def _node_body(spe_ref, p128_ref, p16_ref, h8_ref, wo_ref, wd1_ref, bd1_ref,
               wd2_ref, bd2_ref, wg_ref, s2t_ref, t8_ref, sel_ref,
               ns_ref, vec_ref):
    dh = p128_ref[:, :]
    a16 = p16_ref[:, :]

    lane8 = lax.broadcasted_iota(jnp.int32, (BN, 8), 1)
    p = (spe_ref[:, :] == lane8).astype(jnp.float32)
    h0 = jnp.dot(p, h8_ref[:, :], preferred_element_type=jnp.float32, precision=lax.Precision.HIGHEST)
    h = h0 + jnp.dot(dh, wo_ref[:, :], preferred_element_type=jnp.float32, precision=lax.Precision.HIGHEST)

    t = jnp.dot(h, wd1_ref[:, :], preferred_element_type=jnp.float32, precision=lax.Precision.HIGHEST) \
        + bd1_ref[:, :]
    t = t * jax.nn.sigmoid(t)
    ns_ref[:, :] = (jnp.sum(t * wd2_ref[:, :], axis=1, keepdims=True)
                    + bd2_ref[:, :])

    gate = jnp.dot(h, wg_ref[:, :], preferred_element_type=jnp.float32, precision=lax.Precision.HIGHEST)
    g8 = jnp.dot(gate, s2t_ref[:, :], preferred_element_type=jnp.float32, precision=lax.Precision.HIGHEST)
    g16 = jnp.dot(g8, t8_ref[:, :], preferred_element_type=jnp.float32, precision=lax.Precision.HIGHEST)
    vec8 = jnp.dot(a16 * g16, sel_ref[:, :],
                   preferred_element_type=jnp.float32, precision=lax.Precision.HIGHEST)
    vec_ref[:, :] = vec8[:, :3]


def _tc_node(spe, p128, p16, h8, wo, wd1, bd1, wd2row, bd2s, wg, s2t, t8,
             sel):
    grid = N // BN
    return pl.pallas_call(
        _node_body,
        grid=(grid,),
        in_specs=[
            pl.BlockSpec((BN, 1), lambda i: (i, 0)),
            pl.BlockSpec((BN, D), lambda i: (i, 0)),
            pl.BlockSpec((BN, 16), lambda i: (i, 0)),
            pl.BlockSpec((8, D), lambda i: (0, 0)),
            pl.BlockSpec((D, D), lambda i: (0, 0)),
            pl.BlockSpec((D, 64), lambda i: (0, 0)),
            pl.BlockSpec((1, 64), lambda i: (0, 0)),
            pl.BlockSpec((1, 64), lambda i: (0, 0)),
            pl.BlockSpec((1, 1), lambda i: (0, 0)),
            pl.BlockSpec((D, D), lambda i: (0, 0)),
            pl.BlockSpec((D, 8), lambda i: (0, 0)),
            pl.BlockSpec((8, 16), lambda i: (0, 0)),
            pl.BlockSpec((16, 8), lambda i: (0, 0)),
        ],
        out_specs=[
            pl.BlockSpec((BN, 1), lambda i: (i, 0)),
            pl.BlockSpec((BN, 3), lambda i: (i, 0)),
        ],
        out_shape=[
            jax.ShapeDtypeStruct((N, 1), jnp.float32),
            jax.ShapeDtypeStruct((N, 3), jnp.float32),
        ],
    )(spe, p128, p16, h8, wo, wd1, bd1, wd2row, bd2s, wg, s2t, t8, sel)


def kernel(species, edge_index, edge_attr, edge_vec, W1, W2, b2, We, be,
           Wq, Wk, Wv, Wo, Wd1, bd1, Wd2, bd2, Wg):
    species = species.astype(jnp.int32)
    src = edge_index[0].astype(jnp.int32)
    dst = edge_index[1].astype(jnp.int32)

    pm = jnp.promote_types
    hp = lambda a, b: jnp.dot(a, b, precision=lax.Precision.HIGHEST)
    h5 = hp(jax.nn.silu(W1), W2) + b2
    h8 = jnp.concatenate([h5, jnp.zeros((3, D), jnp.float32)], axis=0)
    q8 = hp(h8, Wq)
    k8 = hp(h8, Wk)
    vm8 = hp(h8, Wv[:, :D])
    s2_5 = hp(h5, Wv[:, 2 * D:])
    s2t = jnp.concatenate([s2_5.T, jnp.zeros((D, 3), jnp.float32)], axis=1)

    wer = We[:NUM_RBF]
    wea = We[NUM_RBF:]
    be_row = be.reshape(1, D)

    u8 = np.zeros((8, 16), np.float32)
    t8 = np.zeros((8, 16), np.float32)
    sel = np.zeros((16, 8), np.float32)
    for c in range(3):
        u8[c, c * 5:c * 5 + 5] = 1.0
        for b in range(5):
            t8[b, c * 5 + b] = 1.0
            sel[c * 5 + b, c] = 1.0
    u8 = jnp.asarray(u8)
    t8 = jnp.asarray(t8)
    sel = jnp.asarray(sel)

    sps, spd = species[src], species[dst]
    msg, amsg = _tc_edge(sps.reshape(E // BE, 1, BE), spd.reshape(E // BE, 1, BE),
                         edge_attr, edge_vec, wer, wea, be_row,
                         q8, k8, vm8, u8, t8)

    dh, a16 = _tc_scatter(dst.reshape(E // BS, 1, BS), msg, amsg)

    ns, vec = _tc_node(species.reshape(N, 1), dh, a16,
                       h8, Wo, Wd1, bd1.reshape(1, 64), Wd2.reshape(1, 64),
                       bd2.reshape(1, 1), Wg, s2t, t8, sel)
    return ns, vec


# speedup vs baseline: 1.1946x; 1.1402x over previous
import functools

import jax
import jax.numpy as jnp
import numpy as np
from jax import lax
from jax.experimental import pallas as pl
from jax.experimental.pallas import tpu as pltpu
from jax.experimental.pallas import tpu_sc as plsc

N = 10000
E = 160000
D = 128
NUM_RBF = 96
NUM_SPECIES = 5
D_EDGE = 16
CUTOFF = 5.0

NPAD = 10240
ROWS_PER_TILE = NPAD // 16

NC, NS, NW = 2, 16, 32
EDGES_PER_TILE = E // NW
CHUNK = 128
N_FULL = EDGES_PER_TILE // CHUNK
TAIL = EDGES_PER_TILE - N_FULL * CHUNK
BOUNCE = CHUNK

BE = 2000
BN = 2000


def _sc_gather_body(src_hbm, dst_hbm, species_hbm, sps_hbm, spd_hbm,
                    spec_v, idx_v, out_v, idx_t, out_t):
    wid = lax.axis_index("s") * NC + lax.axis_index("c")
    base = wid * EDGES_PER_TILE

    pltpu.sync_copy(species_hbm, spec_v)

    def one(edge_hbm, out_hbm, off):
        pltpu.sync_copy(edge_hbm.at[pl.ds(off, CHUNK)], idx_v)
        for j in range(CHUNK // 16):
            idx16 = idx_v[pl.ds(j * 16, 16)]
            out_v[pl.ds(j * 16, 16)] = plsc.load_gather(spec_v, [idx16])
        pltpu.sync_copy(out_v, out_hbm.at[pl.ds(off, CHUNK)])

    for i in range(N_FULL):
        one(src_hbm, sps_hbm, base + i * CHUNK)
        one(dst_hbm, spd_hbm, base + i * CHUNK)

    off = base + N_FULL * CHUNK

    def tail(edge_hbm, out_hbm):
        idx_t[...] = jnp.zeros((16,), jnp.int32)
        pltpu.sync_copy(edge_hbm.at[pl.ds(off, TAIL)], idx_t.at[pl.ds(0, TAIL)])
        out_t[...] = plsc.load_gather(spec_v, [idx_t[...]])
        pltpu.sync_copy(out_t.at[pl.ds(0, TAIL)], out_hbm.at[pl.ds(off, TAIL)])

    tail(src_hbm, sps_hbm)
    tail(dst_hbm, spd_hbm)


def _sc_gather(src, dst, species):
    k = pl.kernel(
        _sc_gather_body,
        mesh=plsc.VectorSubcoreMesh(core_axis_name="c", subcore_axis_name="s"),
        out_type=(jax.ShapeDtypeStruct((E,), jnp.int32),
                  jax.ShapeDtypeStruct((E,), jnp.int32)),
        scratch_types=[
            pltpu.VMEM((N,), jnp.int32),
            pltpu.VMEM((CHUNK,), jnp.int32),
            pltpu.VMEM((CHUNK,), jnp.int32),
            pltpu.VMEM((16,), jnp.int32),
            pltpu.VMEM((16,), jnp.int32),
        ],
    )
    return k(src, dst, species)


def _edge_body(sps_ref, spd_ref, eattr_ref, evec_ref,
               wer_ref, wea_ref, be_ref, q8_ref, k8_ref, vm8_ref, u8_ref,
               t8_ref, msg_ref, amsg_ref):
    sps_row = sps_ref[0, :, :]
    spd_row = spd_ref[0, :, :]
    evec = evec_ref[:, :]
    elen2 = jnp.sum(evec * evec, axis=1, keepdims=True)
    elen = jnp.sqrt(elen2)

    lane96 = lax.broadcasted_iota(jnp.int32, (BE, NUM_RBF), 1)
    centers = lane96.astype(jnp.float32) * (CUTOFF / (NUM_RBF - 1))
    rbf = jnp.exp(-10.0 * (elen - centers) ** 2)

    pre = (jnp.dot(rbf, wer_ref[:, :], preferred_element_type=jnp.float32, precision=lax.Precision.HIGHEST)
           + jnp.dot(eattr_ref[:, :], wea_ref[:, :],
                     preferred_element_type=jnp.float32, precision=lax.Precision.HIGHEST)
           + be_ref[:, :])
    ea = pre * jax.nn.sigmoid(pre)

    sub8 = lax.broadcasted_iota(jnp.int32, (8, BE), 0)
    p_st = (sps_row == sub8).astype(jnp.float32)
    p_dt = (spd_row == sub8).astype(jnp.float32)

    dimnum = (((0,), (0,)), ((), ()))
    qd = lax.dot_general(p_dt, q8_ref[:, :], dimnum,
                         preferred_element_type=jnp.float32, precision=lax.Precision.HIGHEST)
    ks = lax.dot_general(p_st, k8_ref[:, :], dimnum,
                         preferred_element_type=jnp.float32, precision=lax.Precision.HIGHEST)
    vm = lax.dot_general(p_st, vm8_ref[:, :], dimnum,
                         preferred_element_type=jnp.float32, precision=lax.Precision.HIGHEST)

    logit = jnp.sum(ea * qd * ks, axis=1, keepdims=True) / np.sqrt(D)
    cut = jnp.where(elen < CUTOFF,
                    0.5 * (jnp.cos(jnp.pi * elen / CUTOFF) + 1.0), 0.0)
    attn = logit * jax.nn.sigmoid(logit) * cut

    msg_ref[:, :] = vm * ea * attn

    ehat = evec / (elen + 1e-8)
    hat16 = jnp.dot(ehat, u8_ref[:3, :],
                    preferred_element_type=jnp.float32, precision=lax.Precision.HIGHEST)
    p16 = lax.dot_general(p_st, t8_ref[:, :], dimnum,
                          preferred_element_type=jnp.float32, precision=lax.Precision.HIGHEST)
    amsg_ref[:, :] = hat16 * p16 * attn


def _tc_edge(sps, spd, edge_attr, edge_vec, wer, wea, be_row, q8, k8, vm8,
             u8, t8):
    grid = E // BE
    return pl.pallas_call(
        _edge_body,
        grid=(grid,),
        in_specs=[
            pl.BlockSpec((1, 1, BE), lambda i: (i, 0, 0)),
            pl.BlockSpec((1, 1, BE), lambda i: (i, 0, 0)),
            pl.BlockSpec((BE, D_EDGE), lambda i: (i, 0)),
            pl.BlockSpec((BE, 3), lambda i: (i, 0)),
            pl.BlockSpec((NUM_RBF, D), lambda i: (0, 0)),
            pl.BlockSpec((D_EDGE, D), lambda i: (0, 0)),
            pl.BlockSpec((1, D), lambda i: (0, 0)),
            pl.BlockSpec((8, D), lambda i: (0, 0)),
            pl.BlockSpec((8, D), lambda i: (0, 0)),
            pl.BlockSpec((8, D), lambda i: (0, 0)),
            pl.BlockSpec((8, 16), lambda i: (0, 0)),
            pl.BlockSpec((8, 16), lambda i: (0, 0)),
        ],
        out_specs=[
            pl.BlockSpec((BE, D), lambda i: (i, 0)),
            pl.BlockSpec((BE, 16), lambda i: (i, 0)),
        ],
        out_shape=[
            jax.ShapeDtypeStruct((E, D), jnp.float32),
            jax.ShapeDtypeStruct((E, 16), jnp.float32),
        ],
    )(sps, spd, edge_attr, edge_vec, wer, wea, be_row, q8, k8, vm8, u8, t8)


BS = 2000


def _scat_body(dst_ref, msg_ref, amsg_ref, dh_ref, a16_ref):
    pid = pl.program_id(0)

    @pl.when(pid == 0)
    def _():
        dh_ref[...] = jnp.zeros((N, D), jnp.float32)
        a16_ref[...] = jnp.zeros((N, 16), jnp.float32)

    def body(i, _):
        base = i * 8
        for k in range(8):
            d = dst_ref[0, 0, base + k]
            dh_ref[pl.ds(d, 1), :] += msg_ref[pl.ds(base + k, 1), :]
            a16_ref[pl.ds(d, 1), :] += amsg_ref[pl.ds(base + k, 1), :]
        return ()

    lax.fori_loop(0, BS // 8, body, ())


def _tc_scatter(dst3, msg, amsg):
    grid = E // BS
    return pl.pallas_call(
        _scat_body,
        grid=(grid,),
        in_specs=[
            pl.BlockSpec((1, 1, BS), lambda i: (i, 0, 0),
                         memory_space=pltpu.SMEM),
            pl.BlockSpec((BS, D), lambda i: (i, 0)),
            pl.BlockSpec((BS, 16), lambda i: (i, 0)),
        ],
        out_specs=[
            pl.BlockSpec((N, D), lambda i: (0, 0)),
            pl.BlockSpec((N, 16), lambda i: (0, 0)),
        ],
        out_shape=[
            jax.ShapeDtypeStruct((N, D), jnp.float32),
            jax.ShapeDtypeStruct((N, 16), jnp.float32),
        ],
    )(dst3, msg, amsg)


def _node_body(spe_ref, p128_ref, p16_ref, h8_ref, wo_ref, wd1_ref, bd1_ref,
               wd2_ref, bd2_ref, wg_ref, s2t_ref, t8_ref, sel_ref,
               ns_ref, vec_ref):
    dh = p128_ref[:, :]
    a16 = p16_ref[:, :]

    lane8 = lax.broadcasted_iota(jnp.int32, (BN, 8), 1)
    p = (spe_ref[:, :] == lane8).astype(jnp.float32)
    h0 = jnp.dot(p, h8_ref[:, :], preferred_element_type=jnp.float32, precision=lax.Precision.HIGHEST)
    h = h0 + jnp.dot(dh, wo_ref[:, :], preferred_element_type=jnp.float32, precision=lax.Precision.HIGHEST)

    t = jnp.dot(h, wd1_ref[:, :], preferred_element_type=jnp.float32, precision=lax.Precision.HIGHEST) \
        + bd1_ref[:, :]
    t = t * jax.nn.sigmoid(t)
    ns_ref[:, :] = (jnp.sum(t * wd2_ref[:, :], axis=1, keepdims=True)
                    + bd2_ref[:, :])

    gate = jnp.dot(h, wg_ref[:, :], preferred_element_type=jnp.float32, precision=lax.Precision.HIGHEST)
    g8 = jnp.dot(gate, s2t_ref[:, :], preferred_element_type=jnp.float32, precision=lax.Precision.HIGHEST)
    g16 = jnp.dot(g8, t8_ref[:, :], preferred_element_type=jnp.float32, precision=lax.Precision.HIGHEST)
    vec8 = jnp.dot(a16 * g16, sel_ref[:, :],
                   preferred_element_type=jnp.float32, precision=lax.Precision.HIGHEST)
    vec_ref[:, :] = vec8[:, :3]


def _tc_node(spe, p128, p16, h8, wo, wd1, bd1, wd2row, bd2s, wg, s2t, t8,
             sel):
    grid = N // BN
    return pl.pallas_call(
        _node_body,
        grid=(grid,),
        in_specs=[
            pl.BlockSpec((BN, 1), lambda i: (i, 0)),
            pl.BlockSpec((BN, D), lambda i: (i, 0)),
            pl.BlockSpec((BN, 16), lambda i: (i, 0)),
            pl.BlockSpec((8, D), lambda i: (0, 0)),
            pl.BlockSpec((D, D), lambda i: (0, 0)),
            pl.BlockSpec((D, 64), lambda i: (0, 0)),
            pl.BlockSpec((1, 64), lambda i: (0, 0)),
            pl.BlockSpec((1, 64), lambda i: (0, 0)),
            pl.BlockSpec((1, 1), lambda i: (0, 0)),
            pl.BlockSpec((D, D), lambda i: (0, 0)),
            pl.BlockSpec((D, 8), lambda i: (0, 0)),
            pl.BlockSpec((8, 16), lambda i: (0, 0)),
            pl.BlockSpec((16, 8), lambda i: (0, 0)),
        ],
        out_specs=[
            pl.BlockSpec((BN, 1), lambda i: (i, 0)),
            pl.BlockSpec((BN, 3), lambda i: (i, 0)),
        ],
        out_shape=[
            jax.ShapeDtypeStruct((N, 1), jnp.float32),
            jax.ShapeDtypeStruct((N, 3), jnp.float32),
        ],
    )(spe, p128, p16, h8, wo, wd1, bd1, wd2row, bd2s, wg, s2t, t8, sel)


def kernel(species, edge_index, edge_attr, edge_vec, W1, W2, b2, We, be,
           Wq, Wk, Wv, Wo, Wd1, bd1, Wd2, bd2, Wg):
    species = species.astype(jnp.int32)
    src = edge_index[0].astype(jnp.int32)
    dst = edge_index[1].astype(jnp.int32)

    pm = jnp.promote_types
    hp = lambda a, b: jnp.dot(a, b, precision=lax.Precision.HIGHEST)
    h5 = hp(jax.nn.silu(W1), W2) + b2
    h8 = jnp.concatenate([h5, jnp.zeros((3, D), jnp.float32)], axis=0)
    q8 = hp(h8, Wq)
    k8 = hp(h8, Wk)
    vm8 = hp(h8, Wv[:, :D])
    s2_5 = hp(h5, Wv[:, 2 * D:])
    s2t = jnp.concatenate([s2_5.T, jnp.zeros((D, 3), jnp.float32)], axis=1)

    wer = We[:NUM_RBF]
    wea = We[NUM_RBF:]
    be_row = be.reshape(1, D)

    u8 = np.zeros((8, 16), np.float32)
    t8 = np.zeros((8, 16), np.float32)
    sel = np.zeros((16, 8), np.float32)
    for c in range(3):
        u8[c, c * 5:c * 5 + 5] = 1.0
        for b in range(5):
            t8[b, c * 5 + b] = 1.0
            sel[c * 5 + b, c] = 1.0
    u8 = jnp.asarray(u8)
    t8 = jnp.asarray(t8)
    sel = jnp.asarray(sel)

    sps, spd = species[src], species[dst]
    msg, amsg = _tc_edge(sps.reshape(E // BE, 1, BE), spd.reshape(E // BE, 1, BE),
                         edge_attr, edge_vec, wer, wea, be_row,
                         q8, k8, vm8, u8, t8)

    dh, a16 = _tc_scatter(dst.reshape(E // BS, 1, BS), msg, amsg)

    ns, vec = _tc_node(species.reshape(N, 1), dh, a16,
                       h8, Wo, Wd1, bd1.reshape(1, 64), Wd2.reshape(1, 64),
                       bd2.reshape(1, 1), Wg, s2t, t8, sel)
    return ns, vec
